# baseline (speedup 1.0000x reference)
import functools

import jax
import jax.numpy as jnp
from jax import lax
from jax.experimental import pallas as pl
from jax.experimental.pallas import tpu as pltpu

VOCAB_N = 1000000
L_SEQ = 200
D = 64
N_OUT = 1024
BLK = 128
TAIL0 = VOCAB_N // BLK * BLK
TAIL_W = VOCAB_N - TAIL0
MAX_START = TAIL0 - BLK
NCHUNK = 5
CH = L_SEQ // NCHUNK


def _body(idx_smem, idx_hbm, table_ref, wt_hbm, b_hbm, o_ref,
          blocks, idx_v_s, wt_s, b_s, tailbuf, sems, small_sem, tail_sem):
    pltpu.make_async_copy(idx_hbm, idx_v_s, small_sem).start()
    pltpu.make_async_copy(wt_hbm, wt_s, small_sem).start()
    pltpu.make_async_copy(b_hbm, b_s, small_sem).start()
    pltpu.make_async_copy(
        table_ref.at[:, pl.ds(TAIL0, TAIL_W)], tailbuf, tail_sem
    ).start()

    for j in range(L_SEQ):
        start = idx_smem[j] // BLK * BLK
        start = pl.multiple_of(jnp.minimum(start, MAX_START), BLK)
        pltpu.make_async_copy(
            table_ref.at[:, pl.ds(start, BLK)], blocks.at[j],
            sems.at[j // CH],
        ).start()

    pltpu.make_async_copy(idx_hbm, idx_v_s, small_sem).wait()

    idx_v = idx_v_s[...]
    lane = idx_v - jnp.minimum(idx_v // BLK * BLK, MAX_START)
    lanes_iota = lax.broadcasted_iota(jnp.int32, (L_SEQ, 1, BLK), 2)
    cond_m = lanes_iota == lane[:, :, None]
    cond_t = (lanes_iota[:, 0, :] == idx_v - TAIL0).astype(jnp.float32)

    xs = []
    for k in range(NCHUNK):
        for _ in range(CH):
            pltpu.make_async_copy(
                table_ref.at[:, pl.ds(0, BLK)], blocks.at[0], sems.at[k]
            ).wait()
        sl = slice(k * CH, (k + 1) * CH)
        xs.append(jnp.sum(jnp.where(cond_m[sl], blocks[sl], 0.0), axis=2))

    x = jnp.concatenate(xs, axis=0)

    pltpu.make_async_copy(
        table_ref.at[:, pl.ds(TAIL0, TAIL_W)], tailbuf, tail_sem
    ).wait()
    tail_mat = jnp.concatenate(
        [tailbuf[...].T, jnp.zeros((BLK - TAIL_W, D), jnp.float32)], axis=0
    )
    x = x + lax.dot_general(cond_t, tail_mat, (((1,), (0,)), ((), ())),
                            preferred_element_type=jnp.float32)
    x = jnp.where(x >= 0, x, 0.01 * x)

    pltpu.make_async_copy(wt_hbm, wt_s, small_sem).wait()
    pltpu.make_async_copy(b_hbm, b_s, small_sem).wait()
    y = lax.dot_general(x, wt_s[...], (((1,), (0,)), ((), ())),
                        preferred_element_type=jnp.float32)
    o_ref[...] = jax.nn.sigmoid(y + b_s[...])


@functools.cache
def _fused_call():
    return pl.pallas_call(
        _body,
        grid=(),
        in_specs=[
            pl.BlockSpec(memory_space=pltpu.SMEM),
            pl.BlockSpec(memory_space=pl.ANY),
            pl.BlockSpec(memory_space=pl.ANY),
            pl.BlockSpec(memory_space=pl.ANY),
            pl.BlockSpec(memory_space=pl.ANY),
        ],
        out_shape=jax.ShapeDtypeStruct((L_SEQ, N_OUT), jnp.float32),
        scratch_shapes=[
            pltpu.VMEM((L_SEQ, D, BLK), jnp.float32),
            pltpu.VMEM((L_SEQ, 1), jnp.int32),
            pltpu.VMEM((D, N_OUT), jnp.float32),
            pltpu.VMEM((1, N_OUT), jnp.float32),
            pltpu.VMEM((D, TAIL_W), jnp.float32),
            pltpu.SemaphoreType.DMA((NCHUNK,)),
            pltpu.SemaphoreType.DMA,
            pltpu.SemaphoreType.DMA,
        ],
    )


@jax.jit
def kernel(inputs, emb_table, W, b):
    idx = inputs[0].astype(jnp.int32)
    return _fused_call()(
        idx, idx.reshape(L_SEQ, 1), emb_table.T, W.T, b.reshape(1, N_OUT)
    )


# speedup vs baseline: 1.1919x; 1.1919x over previous
import functools

import jax
import jax.numpy as jnp
from jax import lax
from jax.experimental import pallas as pl
from jax.experimental.pallas import tpu as pltpu

VOCAB_N = 1000000
L_SEQ = 200
D = 64
N_OUT = 1024
BLK = 128
TAIL0 = VOCAB_N // BLK * BLK
TAIL_W = VOCAB_N - TAIL0
MAX_START = TAIL0 - BLK
NCHUNK = 5
CH = L_SEQ // NCHUNK


def _body(idx_smem, idx_vmem, table_ref, wt_ref, b_ref, o_ref,
          blocks, tailbuf, sems, tail_sem):
    pltpu.make_async_copy(
        table_ref.at[:, pl.ds(TAIL0, TAIL_W)], tailbuf, tail_sem
    ).start()

    for j in range(L_SEQ):
        start = idx_smem[j] // BLK * BLK
        start = pl.multiple_of(jnp.minimum(start, MAX_START), BLK)
        pltpu.make_async_copy(
            table_ref.at[:, pl.ds(start, BLK)], blocks.at[j],
            sems.at[j // CH],
        ).start()

    idx_v = idx_vmem[...]
    lane = idx_v - jnp.minimum(idx_v // BLK * BLK, MAX_START)
    lanes_iota = lax.broadcasted_iota(jnp.int32, (L_SEQ, 1, BLK), 2)
    cond_m = lanes_iota == lane[:, :, None]
    cond_t = (lanes_iota[:, 0, :] == idx_v - TAIL0).astype(jnp.float32)

    xs = []
    for k in range(NCHUNK):
        pltpu.make_async_copy(
            blocks.at[pl.ds(k * CH, CH)], blocks.at[pl.ds(k * CH, CH)],
            sems.at[k],
        ).wait()
        sl = slice(k * CH, (k + 1) * CH)
        xs.append(jnp.sum(jnp.where(cond_m[sl], blocks[sl], 0.0), axis=2))

    x = jnp.concatenate(xs, axis=0)

    pltpu.make_async_copy(
        table_ref.at[:, pl.ds(TAIL0, TAIL_W)], tailbuf, tail_sem
    ).wait()
    tail_mat = jnp.concatenate(
        [tailbuf[...].T, jnp.zeros((BLK - TAIL_W, D), jnp.float32)], axis=0
    )
    x = x + lax.dot_general(cond_t, tail_mat, (((1,), (0,)), ((), ())),
                            preferred_element_type=jnp.float32)
    x = jnp.where(x >= 0, x, 0.01 * x)
    y = lax.dot_general(x, wt_ref[...], (((1,), (0,)), ((), ())),
                        preferred_element_type=jnp.float32)
    o_ref[...] = jax.nn.sigmoid(y + b_ref[...])


@functools.cache
def _fused_call():
    return pl.pallas_call(
        _body,
        grid=(),
        in_specs=[
            pl.BlockSpec(memory_space=pltpu.SMEM),
            pl.BlockSpec(memory_space=pltpu.VMEM),
            pl.BlockSpec(memory_space=pl.ANY),
            pl.BlockSpec(memory_space=pltpu.VMEM),
            pl.BlockSpec(memory_space=pltpu.VMEM),
        ],
        out_shape=jax.ShapeDtypeStruct((L_SEQ, N_OUT), jnp.float32),
        scratch_shapes=[
            pltpu.VMEM((L_SEQ, D, BLK), jnp.float32),
            pltpu.VMEM((D, TAIL_W), jnp.float32),
            pltpu.SemaphoreType.DMA((NCHUNK,)),
            pltpu.SemaphoreType.DMA,
        ],
    )


@jax.jit
def kernel(inputs, emb_table, W, b):
    idx = inputs[0].astype(jnp.int32)
    return _fused_call()(
        idx, idx.reshape(L_SEQ, 1), emb_table.T, W.T, b.reshape(1, N_OUT)
    )
